# baseline (speedup 1.0000x reference)
import functools

import jax
import jax.numpy as jnp
from jax import lax
from jax.experimental import pallas as pl
from jax.experimental.pallas import tpu as pltpu
from jax.experimental.pallas import tpu_sc as plsc

VOCAB = 1000000
EMBED_DIM = 64
OUTPUT_DIM = 16
SEQ = 200
BATCH = 4096

NC = 2
NS = 16
NW = NC * NS
BPW = BATCH // NW
HALF = SEQ // 2
HPAD = 104
PAIRS = BPW // 2
LANES = 16
DGRP = EMBED_DIM // LANES


def _fasttext_kernel(table, idx, wt, bias, out, idx_v, rows_v, w_v, b_v,
                     outb_v, sem0, sem1):
    wid = lax.axis_index("s") * NC + lax.axis_index("c")
    base = wid * BPW

    pltpu.sync_copy(idx.at[pl.ds(base, BPW)], idx_v)
    pltpu.sync_copy(wt, w_v)
    pltpu.sync_copy(bias, b_v)

    sems = (sem0, sem1)

    def start_gather(b, db):
        for h in range(2):
            pltpu.make_async_copy(
                table.at[idx_v.at[b, h]], rows_v.at[db, h], sems[db]
            ).start()

    def wait_gather(db):
        for h in range(2):
            pltpu.make_async_copy(
                table.at[idx_v.at[0, h]], rows_v.at[db, h], sems[db]
            ).wait()

    zero = jnp.zeros((LANES,), jnp.float32)

    def process(b, db):
        def red(t, accs):
            new = []
            for g in range(DGRP):
                a = accs[g]
                a = a + rows_v[db, 0, t, pl.ds(g * LANES, LANES)]
                a = a + rows_v[db, 1, t, pl.ds(g * LANES, LANES)]
                new.append(a)
            return tuple(new)

        accs = lax.fori_loop(0, HALF, red, (zero,) * DGRP, unroll=4)

        o_v = zero
        for d in range(EMBED_DIM):
            o_v = o_v + accs[d // LANES][d % LANES] * w_v[d]
        outb_v[b] = o_v * (1.0 / SEQ) + b_v[:]

    start_gather(0, 0)

    def pair(p, carry):
        b0 = 2 * p
        start_gather(b0 + 1, 1)
        wait_gather(0)
        process(b0, 0)

        @pl.when(p < PAIRS - 1)
        def _():
            start_gather(b0 + 2, 0)

        wait_gather(1)
        process(b0 + 1, 1)
        return carry

    lax.fori_loop(0, PAIRS, pair, 0)

    pltpu.sync_copy(outb_v, out.at[pl.ds(base, BPW)])


@jax.jit
def _fasttext(table, idx, wt, bias):
    mesh = plsc.VectorSubcoreMesh(
        core_axis_name="c", subcore_axis_name="s", num_cores=NC,
        num_subcores=NS)
    return pl.kernel(
        _fasttext_kernel,
        out_type=jax.ShapeDtypeStruct((BATCH, OUTPUT_DIM), jnp.float32),
        mesh=mesh,
        compiler_params=pltpu.CompilerParams(use_tc_tiling_on_sc=False),
        scratch_types=[
            pltpu.VMEM((BPW, 2, HPAD), jnp.int32),
            pltpu.VMEM((2, 2, HPAD, EMBED_DIM), jnp.float32),
            pltpu.VMEM((EMBED_DIM, OUTPUT_DIM), jnp.float32),
            pltpu.VMEM((OUTPUT_DIM,), jnp.float32),
            pltpu.VMEM((BPW, OUTPUT_DIM), jnp.float32),
            pltpu.SemaphoreType.DMA,
            pltpu.SemaphoreType.DMA,
        ],
    )(table, idx, wt, bias)


def kernel(text, emb_table, fc_w, fc_b):
    idx = text.T.astype(jnp.int32).reshape(BATCH, 2, HALF)
    idx = jnp.pad(idx, ((0, 0), (0, 0), (0, HPAD - HALF)))
    return _fasttext(emb_table, idx, fc_w.T.astype(jnp.float32),
                     fc_b.astype(jnp.float32))


# speedup vs baseline: 1.0031x; 1.0031x over previous
import functools

import jax
import jax.numpy as jnp
from jax import lax
from jax.experimental import pallas as pl
from jax.experimental.pallas import tpu as pltpu
from jax.experimental.pallas import tpu_sc as plsc

VOCAB = 1000000
EMBED_DIM = 64
OUTPUT_DIM = 16
SEQ = 200
BATCH = 4096

NC = 2
NS = 16
NW = NC * NS
BPW = BATCH // NW
HALF = SEQ // 2
HPAD = 104
PAIRS = BPW // 2
LANES = 16
DGRP = EMBED_DIM // LANES


NBUF = 4


def _fasttext_kernel(table, idx, wt, bias, out, idx_v, rows_v, w_v, b_v,
                     outb_v, *sems):
    wid = lax.axis_index("s") * NC + lax.axis_index("c")
    base = wid * BPW

    pltpu.sync_copy(idx.at[pl.ds(base, BPW)], idx_v)
    pltpu.sync_copy(wt, w_v)
    pltpu.sync_copy(bias, b_v)

    def start_gather(b, db):
        for h in range(2):
            pltpu.make_async_copy(
                table.at[idx_v.at[b, h]], rows_v.at[db, h], sems[db]
            ).start()

    def wait_gather(db):
        for h in range(2):
            pltpu.make_async_copy(
                table.at[idx_v.at[0, h]], rows_v.at[db, h], sems[db]
            ).wait()

    zero = jnp.zeros((LANES,), jnp.float32)

    def process(b, db):
        def red(t, accs):
            new = []
            for g in range(DGRP):
                a = accs[g]
                a = a + rows_v[db, 0, t, pl.ds(g * LANES, LANES)]
                a = a + rows_v[db, 1, t, pl.ds(g * LANES, LANES)]
                new.append(a)
            return tuple(new)

        accs = lax.fori_loop(0, HALF, red, (zero,) * DGRP, unroll=4)

        o_v = zero
        for d in range(EMBED_DIM):
            o_v = o_v + accs[d // LANES][d % LANES] * w_v[d]
        outb_v[b] = o_v * (1.0 / SEQ) + b_v[:]

    for j in range(NBUF - 1):
        start_gather(j, j)

    def group(q, carry):
        for j in range(NBUF):
            b = NBUF * q + j

            @pl.when(b + NBUF - 1 < BPW)
            def _():
                start_gather(b + NBUF - 1, (j + NBUF - 1) % NBUF)

            wait_gather(j)
            process(b, j)
        return carry

    lax.fori_loop(0, BPW // NBUF, group, 0)

    pltpu.sync_copy(outb_v, out.at[pl.ds(base, BPW)])


@jax.jit
def _fasttext(table, idx, wt, bias):
    mesh = plsc.VectorSubcoreMesh(
        core_axis_name="c", subcore_axis_name="s", num_cores=NC,
        num_subcores=NS)
    return pl.kernel(
        _fasttext_kernel,
        out_type=jax.ShapeDtypeStruct((BATCH, OUTPUT_DIM), jnp.float32),
        mesh=mesh,
        compiler_params=pltpu.CompilerParams(use_tc_tiling_on_sc=False),
        scratch_types=[
            pltpu.VMEM((BPW, 2, HPAD), jnp.int32),
            pltpu.VMEM((NBUF, 2, HPAD, EMBED_DIM), jnp.float32),
            pltpu.VMEM((EMBED_DIM, OUTPUT_DIM), jnp.float32),
            pltpu.VMEM((OUTPUT_DIM,), jnp.float32),
            pltpu.VMEM((BPW, OUTPUT_DIM), jnp.float32),
        ] + [pltpu.SemaphoreType.DMA] * NBUF,
    )(table, idx, wt, bias)


def kernel(text, emb_table, fc_w, fc_b):
    idx = text.T.astype(jnp.int32).reshape(BATCH, 2, HALF)
    idx = jnp.pad(idx, ((0, 0), (0, 0), (0, HPAD - HALF)))
    return _fasttext(emb_table, idx, fc_w.T.astype(jnp.float32),
                     fc_b.astype(jnp.float32))


# speedup vs baseline: 1.7793x; 1.7737x over previous
import jax
import jax.numpy as jnp
from jax import lax
from jax.experimental import pallas as pl
from jax.experimental.pallas import tpu as pltpu
from jax.experimental.pallas import tpu_sc as plsc

VOCAB = 1000000
EMBED_DIM = 64
OUTPUT_DIM = 16
SEQ = 200
BATCH = 4096

NC = 2
NS = 16
NW = NC * NS
BPW = BATCH // NW
LANES = 16
DGRP = EMBED_DIM // LANES

T = 5
NBUF = 2
CHUNKS = SEQ // T
CROWS = T * BPW


def _fasttext_kernel(text, table, wt, bias, out, idx_v, idx1_v, rows_v,
                     acc_v, w_v, b_v, outb_v, *sems):
    wid = lax.axis_index("s") * NC + lax.axis_index("c")
    base = wid * BPW

    pltpu.sync_copy(text.at[:, pl.ds(base, BPW)], idx_v)
    pltpu.sync_copy(wt, w_v)
    pltpu.sync_copy(bias, b_v)

    zero = jnp.zeros((LANES,), jnp.float32)

    def zacc(b, carry):
        for g in range(DGRP):
            acc_v[b, pl.ds(g * LANES, LANES)] = zero
        return carry

    lax.fori_loop(0, BPW, zacc, 0, unroll=8)

    def start_gather(c, db):
        for r in range(T):
            for g in range(BPW // LANES):
                idx1_v[db, pl.ds(r * BPW + g * LANES, LANES)] = (
                    idx_v[c * T + r, pl.ds(g * LANES, LANES)])
        pltpu.make_async_copy(
            table.at[idx1_v.at[db]], rows_v.at[db], sems[db],
        ).start()

    def wait_gather(db):
        pltpu.make_async_copy(
            table.at[idx1_v.at[db]], rows_v.at[db], sems[db],
        ).wait()

    def accumulate(db):
        for tt in range(T):
            def body(b, carry):
                for g in range(DGRP):
                    v = rows_v[db, tt * BPW + b, pl.ds(g * LANES, LANES)]
                    plsc.addupdate(acc_v.at[b, pl.ds(g * LANES, LANES)], v)
                return carry

            lax.fori_loop(0, BPW, body, 0, unroll=4)

    for j in range(NBUF - 1):
        start_gather(j, j)

    def group(q, carry):
        for j in range(NBUF):
            c = NBUF * q + j

            @pl.when(c + NBUF - 1 < CHUNKS)
            def _():
                start_gather(c + NBUF - 1, (j + NBUF - 1) % NBUF)

            wait_gather(j)
            accumulate(j)
        return carry

    lax.fori_loop(0, CHUNKS // NBUF, group, 0)

    b_row = b_v[:]

    def fc(b, carry):
        accs = [acc_v[b, pl.ds(g * LANES, LANES)] for g in range(DGRP)]
        o_v = zero
        for d in range(EMBED_DIM):
            o_v = o_v + accs[d // LANES][d % LANES] * w_v[d]
        outb_v[b] = o_v * (1.0 / SEQ) + b_row
        return carry

    lax.fori_loop(0, BPW, fc, 0)

    pltpu.sync_copy(outb_v, out.at[pl.ds(base, BPW)])


@jax.jit
def _fasttext(text, table, wt, bias):
    mesh = plsc.VectorSubcoreMesh(
        core_axis_name="c", subcore_axis_name="s", num_cores=NC,
        num_subcores=NS)
    return pl.kernel(
        _fasttext_kernel,
        out_type=jax.ShapeDtypeStruct((BATCH, OUTPUT_DIM), jnp.float32),
        mesh=mesh,
        compiler_params=pltpu.CompilerParams(use_tc_tiling_on_sc=False),
        scratch_types=[
            pltpu.VMEM((SEQ, BPW), jnp.int32),
            pltpu.VMEM((NBUF, CROWS), jnp.int32),
            pltpu.VMEM((NBUF, CROWS, EMBED_DIM), jnp.float32),
            pltpu.VMEM((BPW, EMBED_DIM), jnp.float32),
            pltpu.VMEM((EMBED_DIM, OUTPUT_DIM), jnp.float32),
            pltpu.VMEM((OUTPUT_DIM,), jnp.float32),
            pltpu.VMEM((BPW, OUTPUT_DIM), jnp.float32),
        ] + [pltpu.SemaphoreType.DMA] * NBUF,
    )(text, table, wt, bias)


def kernel(text, emb_table, fc_w, fc_b):
    return _fasttext(text.astype(jnp.int32), emb_table,
                     fc_w.T.astype(jnp.float32), fc_b.astype(jnp.float32))
